# baseline (speedup 1.0000x reference)
import functools

import jax
import jax.numpy as jnp
from jax import lax
from jax.experimental import pallas as pl
from jax.experimental.pallas import tpu as pltpu
from jax.experimental.pallas import tpu_sc as plsc

B = 16384
H = 128
EPS = 1e-12

NC = 2
NS = 16
NW = NC * NS
ROWS_PER_W = B // NW
CHUNK = 128
NCHUNK = ROWS_PER_W // CHUNK
L = 16
VPR = H // L


def _rsqrt_vec(v):
    i = lax.bitcast_convert_type(v, jnp.int32)
    i = jnp.int32(0x5F3759DF) - lax.shift_right_logical(i, 1)
    y = lax.bitcast_convert_type(i, jnp.float32)
    half = v * 0.5
    for _ in range(4):
        y = y * (1.5 - half * y * y)
    return y


def _allreduce_sum(v):
    lanes = lax.iota(jnp.int32, L)
    for k in (8, 4, 2, 1):
        perm = v.at[lanes ^ k].get(mode="promise_in_bounds",
                                   unique_indices=True)
        v = v + perm
    return v


_mesh = plsc.VectorSubcoreMesh(core_axis_name="c", subcore_axis_name="s")


@functools.partial(
    pl.kernel,
    mesh=_mesh,
    out_type=jax.ShapeDtypeStruct((B, H), jnp.float32),
    scratch_types=[
        pltpu.VMEM((4, CHUNK), jnp.int32),
        pltpu.VMEM((CHUNK, H), jnp.float32),
        pltpu.VMEM((CHUNK, H), jnp.float32),
        pltpu.VMEM((CHUNK, H), jnp.float32),
        pltpu.VMEM((CHUNK, H), jnp.float32),
        pltpu.VMEM((H,), jnp.float32),
        pltpu.VMEM((H,), jnp.float32),
        pltpu.SemaphoreType.DMA,
    ],
)
def _embed_ln(xT, syn, pos, sen, lem, gamma, beta, out,
              idx_v, b0, b1, b2, b3, g_v, be_v, sem):
    wid = lax.axis_index("s") * NC + lax.axis_index("c")
    base = wid * ROWS_PER_W
    pltpu.sync_copy(gamma, g_v)
    pltpu.sync_copy(beta, be_v)

    def chunk_body(c, carry):
        row0 = base + c * CHUNK
        for t in range(4):
            pltpu.sync_copy(xT.at[t, pl.ds(row0, CHUNK)], idx_v.at[t])
        copies = [
            pltpu.async_copy(tab.at[idx_v.at[t]], buf, sem)
            for t, (tab, buf) in enumerate(
                ((syn, b0), (pos, b1), (sen, b2), (lem, b3)))
        ]
        for cp in copies:
            cp.wait()

        def row_body(r, rcarry):
            accs = []
            for j in range(VPR):
                sl = pl.ds(j * L, L)
                accs.append(b0[r, sl] + b1[r, sl] + b2[r, sl] + b3[r, sl])
            s = accs[0]
            for j in range(1, VPR):
                s = s + accs[j]
            mean = _allreduce_sum(s) * (1.0 / H)
            ds = []
            sq = None
            for j in range(VPR):
                d = accs[j] - mean
                ds.append(d)
                sq = d * d if sq is None else sq + d * d
            var = _allreduce_sum(sq) * (1.0 / H)
            rinv = _rsqrt_vec(var + EPS)
            for j in range(VPR):
                sl = pl.ds(j * L, L)
                b0[r, sl] = ds[j] * rinv * g_v[sl] + be_v[sl]
            return rcarry

        lax.fori_loop(0, CHUNK, row_body, 0)
        pltpu.sync_copy(b0, out.at[pl.ds(row0, CHUNK)])
        return carry

    lax.fori_loop(0, NCHUNK, chunk_body, 0)


def kernel(x, syn_table, lemma_table, pos_table, sense_table, gamma, beta):
    xT = x.T
    return _embed_ln(xT, syn_table, pos_table, sense_table, lemma_table,
                     gamma, beta)


# speedup vs baseline: 2.2251x; 2.2251x over previous
import functools

import jax
import jax.numpy as jnp
from jax import lax
from jax.experimental import pallas as pl
from jax.experimental.pallas import tpu as pltpu
from jax.experimental.pallas import tpu_sc as plsc

B = 16384
H = 128
EPS = 1e-12

NC = 2
NS = 16
NW = NC * NS
ROWS_PER_W = B // NW
CHUNK = 128
NCHUNK = ROWS_PER_W // CHUNK
L = 16
VPR = H // L


def _rsqrt_vec(v):
    i = lax.bitcast_convert_type(v, jnp.int32)
    i = jnp.int32(0x5F3759DF) - lax.shift_right_logical(i, 1)
    y = lax.bitcast_convert_type(i, jnp.float32)
    half = v * 0.5
    for _ in range(3):
        y = y * (1.5 - half * y * y)
    return y


def _allreduce_sum(v, lanes):
    for k in (8, 4, 2, 1):
        perm = v.at[lanes ^ k].get(mode="promise_in_bounds",
                                   unique_indices=True)
        v = v + perm
    return v


_mesh = plsc.VectorSubcoreMesh(core_axis_name="c", subcore_axis_name="s")


@functools.partial(
    pl.kernel,
    mesh=_mesh,
    out_type=jax.ShapeDtypeStruct((B, H), jnp.float32),
    scratch_types=[
        pltpu.VMEM((4, ROWS_PER_W), jnp.int32),
        pltpu.VMEM((CHUNK, H), jnp.float32),
        pltpu.VMEM((CHUNK, H), jnp.float32),
        pltpu.VMEM((CHUNK, H), jnp.float32),
        pltpu.VMEM((CHUNK, H), jnp.float32),
        pltpu.VMEM((H,), jnp.float32),
        pltpu.VMEM((H,), jnp.float32),
        pltpu.SemaphoreType.DMA,
        pltpu.SemaphoreType.DMA,
        pltpu.SemaphoreType.DMA,
        pltpu.SemaphoreType.DMA,
    ],
)
def _embed_ln(xT, syn, pos, sen, lem, gamma, beta, out,
              idx_v, ga, gb, oa, ob, g_v, be_v,
              sem_ga, sem_gb, sem_oa, sem_ob):
    wid = lax.axis_index("s") * NC + lax.axis_index("c")
    base = wid * ROWS_PER_W
    pltpu.sync_copy(gamma, g_v)
    pltpu.sync_copy(beta, be_v)
    pltpu.sync_copy(xT.at[:, pl.ds(base, ROWS_PER_W)], idx_v)

    tables = (syn, pos, sen, lem)
    gbufs = (ga, gb)
    obufs = (oa, ob)
    gsems = (sem_ga, sem_gb)
    osems = (sem_oa, sem_ob)

    zero = jnp.zeros((L,), jnp.float32)

    def zero_buf(buf):
        def zbody(r, carry):
            for j in range(2 * VPR):
                buf[2 * r + j // VPR, pl.ds((j % VPR) * L, L)] = zero
            return carry
        lax.fori_loop(0, CHUNK // 2, zbody, 0)

    def fire_gathers(c):
        p = c % 2
        return [
            pltpu.async_copy(
                tables[t].at[idx_v.at[t, pl.ds(c * CHUNK, CHUNK)]],
                gbufs[p], gsems[p], add=True)
            for t in range(4)
        ]

    lanes = lax.iota(jnp.int32, L)
    gvs = [g_v[pl.ds(j * L, L)] for j in range(VPR)]
    bevs = [be_v[pl.ds(j * L, L)] for j in range(VPR)]

    def compute_chunk(gbuf, obuf):
        def row_body(r2, carry):
            for rr in range(2):
                r = 2 * r2 + rr
                accs = [gbuf[r, pl.ds(j * L, L)] for j in range(VPR)]
                s = accs[0]
                for j in range(1, VPR):
                    s = s + accs[j]
                sq = accs[0] * accs[0]
                for j in range(1, VPR):
                    sq = sq + accs[j] * accs[j]
                s = _allreduce_sum(s, lanes)
                sq = _allreduce_sum(sq, lanes)
                mean = s * (1.0 / H)
                var = sq * (1.0 / H) - mean * mean
                rinv = _rsqrt_vec(var + EPS)
                t0 = mean * rinv
                for j in range(VPR):
                    obuf[r, pl.ds(j * L, L)] = (
                        (accs[j] * rinv - t0) * gvs[j] + bevs[j])
            return carry
        lax.fori_loop(0, CHUNK // 2, row_body, 0)

    zero_buf(ga)
    g_copies = {0: fire_gathers(0)}
    zero_buf(gb)
    g_copies[1] = fire_gathers(1)
    o_copies = {}

    for c in range(NCHUNK):
        p = c % 2
        for cp in g_copies.pop(c):
            cp.wait()
        if c >= 2:
            o_copies.pop(c - 2).wait()
        compute_chunk(gbufs[p], obufs[p])
        o_copies[c] = pltpu.async_copy(
            obufs[p], out.at[pl.ds(base + c * CHUNK, CHUNK)], osems[p])
        if c + 2 < NCHUNK:
            zero_buf(gbufs[p])
            g_copies[c + 2] = fire_gathers(c + 2)

    for c in (NCHUNK - 2, NCHUNK - 1):
        o_copies.pop(c).wait()


def kernel(x, syn_table, lemma_table, pos_table, sense_table, gamma, beta):
    xT = x.T
    return _embed_ln(xT, syn_table, pos_table, sense_table, lemma_table,
                     gamma, beta)
